# baseline (speedup 1.0000x reference)
import functools

import jax
import jax.numpy as jnp
from jax import lax
from jax.experimental import pallas as pl
from jax.experimental.pallas import tpu as pltpu
from jax.experimental.pallas import tpu_sc as plsc

E = 8
K = 2
BN = 1024


def _mm_body(x_ref, w_ref, b_ref, out_ref):
    out_ref[:, :] = lax.dot_general(
        w_ref[:, :], x_ref[:, :],
        dimension_numbers=(((1,), (1,)), ((), ())),
        preferred_element_type=jnp.float32,
    ) + b_ref[:, :]


def _logits_t(x, W, b):
    n, d = x.shape
    grid = n // BN
    return pl.pallas_call(
        _mm_body,
        grid=(grid,),
        in_specs=[
            pl.BlockSpec((BN, d), lambda i: (i, 0)),
            pl.BlockSpec((E, d), lambda i: (0, 0)),
            pl.BlockSpec((E, 1), lambda i: (0, 0)),
        ],
        out_specs=pl.BlockSpec((E, BN), lambda i: (0, i)),
        out_shape=jax.ShapeDtypeStruct((E, n), jnp.float32),
        compiler_params=pltpu.CompilerParams(
            dimension_semantics=("parallel",),
        ),
    )(x, W, b.reshape(E, 1))


def _make_router(n):
    nw = 32
    tpw = n // nw

    @functools.partial(
        pl.kernel,
        out_type=[
            jax.ShapeDtypeStruct((K, n), jnp.float32),
            jax.ShapeDtypeStruct((K, n), jnp.int32),
        ],
        mesh=plsc.VectorSubcoreMesh(core_axis_name="c", subcore_axis_name="s"),
        scratch_types=[
            pltpu.VMEM((E, tpw), jnp.float32),
            pltpu.VMEM((K, tpw), jnp.float32),
            pltpu.VMEM((K, tpw), jnp.int32),
        ],
    )
    def router(logits_hbm, gates_hbm, idx_hbm, lv, gv, iv):
        wid = lax.axis_index("s") * 2 + lax.axis_index("c")
        base = wid * tpw
        pltpu.sync_copy(logits_hbm.at[:, pl.ds(base, tpw)], lv)

        neg = jnp.full((16,), -1e30, jnp.float32)

        def body(g, carry):
            t = g * 16
            v = [lv[e, pl.ds(t, 16)] for e in range(E)]
            m1 = v[0]
            a1 = jnp.zeros((16,), jnp.int32)
            for e in range(1, E):
                gt = v[e] > m1
                m1 = jnp.where(gt, v[e], m1)
                a1 = jnp.where(gt, jnp.full((16,), e, jnp.int32), a1)
            m2 = neg
            a2 = jnp.zeros((16,), jnp.int32)
            for e in range(E):
                ev = jnp.full((16,), e, jnp.int32)
                cand = jnp.where(a1 == ev, neg, v[e])
                gt = cand > m2
                m2 = jnp.where(gt, cand, m2)
                a2 = jnp.where(gt, ev, a2)
            s = jnp.exp(m2 - m1)
            inv = 1.0 / (1.0 + s)
            g1 = inv
            g2 = s * inv
            gv[0, pl.ds(t, 16)] = g1
            gv[1, pl.ds(t, 16)] = g2
            iv[0, pl.ds(t, 16)] = a1
            iv[1, pl.ds(t, 16)] = a2
            return carry

        lax.fori_loop(0, tpw // 16, body, 0)
        pltpu.sync_copy(gv, gates_hbm.at[:, pl.ds(base, tpw)])
        pltpu.sync_copy(iv, idx_hbm.at[:, pl.ds(base, tpw)])

    return router


def kernel(x, W, b):
    n = x.shape[0]
    logits_t = _logits_t(x, W, b)
    gates_t, idx_t = _make_router(n)(logits_t)
    return gates_t.T, idx_t.T


# speedup vs baseline: 1.1739x; 1.1739x over previous
import functools

import jax
import jax.numpy as jnp
from jax import lax
from jax.experimental import pallas as pl
from jax.experimental.pallas import tpu as pltpu
from jax.experimental.pallas import tpu_sc as plsc

E = 8
K = 2
BN = 4096


def _mm_body(x_ref, w_ref, b_ref, out_ref):
    out_ref[:, :] = lax.dot_general(
        w_ref[:, :], x_ref[:, :],
        dimension_numbers=(((1,), (1,)), ((), ())),
        preferred_element_type=jnp.float32,
    ) + b_ref[:, :]


def _logits_t(x, W, b):
    n, d = x.shape
    grid = n // BN
    return pl.pallas_call(
        _mm_body,
        grid=(grid,),
        in_specs=[
            pl.BlockSpec((BN, d), lambda i: (i, 0)),
            pl.BlockSpec((E, d), lambda i: (0, 0)),
            pl.BlockSpec((E, 1), lambda i: (0, 0)),
        ],
        out_specs=pl.BlockSpec((E, BN), lambda i: (0, i)),
        out_shape=jax.ShapeDtypeStruct((E, n), jnp.float32),
        compiler_params=pltpu.CompilerParams(
            dimension_semantics=("parallel",),
        ),
    )(x, W, b.reshape(E, 1))


def _make_router(n):
    nw = 32
    tpw = n // nw

    @functools.partial(
        pl.kernel,
        out_type=[
            jax.ShapeDtypeStruct((K, n), jnp.float32),
            jax.ShapeDtypeStruct((K, n), jnp.int32),
        ],
        mesh=plsc.VectorSubcoreMesh(core_axis_name="c", subcore_axis_name="s"),
        scratch_types=[
            pltpu.VMEM((E, tpw), jnp.float32),
            pltpu.VMEM((K, tpw), jnp.float32),
            pltpu.VMEM((K, tpw), jnp.int32),
        ],
    )
    def router(logits_hbm, gates_hbm, idx_hbm, lv, gv, iv):
        wid = lax.axis_index("s") * 2 + lax.axis_index("c")
        base = wid * tpw
        pltpu.sync_copy(logits_hbm.at[:, pl.ds(base, tpw)], lv)

        neg = jnp.full((16,), -1e30, jnp.float32)

        def body(g, carry):
            t = g * 16
            v = [lv[e, pl.ds(t, 16)] for e in range(E)]
            m1 = v[0]
            a1 = jnp.zeros((16,), jnp.int32)
            for e in range(1, E):
                gt = v[e] > m1
                m1 = jnp.where(gt, v[e], m1)
                a1 = jnp.where(gt, jnp.full((16,), e, jnp.int32), a1)
            m2 = neg
            a2 = jnp.zeros((16,), jnp.int32)
            for e in range(E):
                ev = jnp.full((16,), e, jnp.int32)
                cand = jnp.where(a1 == ev, neg, v[e])
                gt = cand > m2
                m2 = jnp.where(gt, cand, m2)
                a2 = jnp.where(gt, ev, a2)
            s = jnp.exp(m2 - m1)
            inv = 1.0 / (1.0 + s)
            g1 = inv
            g2 = s * inv
            gv[0, pl.ds(t, 16)] = g1
            gv[1, pl.ds(t, 16)] = g2
            iv[0, pl.ds(t, 16)] = a1
            iv[1, pl.ds(t, 16)] = a2
            return carry

        lax.fori_loop(0, tpw // 16, body, 0)
        pltpu.sync_copy(gv, gates_hbm.at[:, pl.ds(base, tpw)])
        pltpu.sync_copy(iv, idx_hbm.at[:, pl.ds(base, tpw)])

    return router


def kernel(x, W, b):
    n = x.shape[0]
    logits_t = _logits_t(x, W, b)
    gates_t, idx_t = _make_router(n)(logits_t)
    return gates_t.T, idx_t.T
